# baseline (speedup 1.0000x reference)
import jax
import jax.numpy as jnp
from jax.experimental import pallas as pl
from jax.experimental.pallas import tpu as pltpu

_TM = 32768
_CK = 4096
_PADS = (2, 0, 1, 1, 1, 1, 0, 2)


def _ru(x, m):
    return (x + m - 1) // m * m


def _make_body(tm, ck, taps, widths, origins, couts, cins):
    n_layers = len(taps)

    def body(*refs):
        xa, xb, ma, mb = refs[0:4]
        out = refs[4 + 3 * n_layers]
        scratch = refs[5 + 3 * n_layers:]
        xcat, mcat = scratch[0], scratch[1]
        acts = scratch[2:2 + n_layers - 1]
        (st_big0, st_big1, st_map0, st_map1,
         part0, part1) = scratch[2 + n_layers - 1:]

        xcat[:, :tm] = xa[...]
        xcat[:, tm:] = xb[...]
        mcat[:, :tm] = ma[...]
        mcat[:, tm:] = mb[...]

        srcs = [xcat] + list(acts)
        for l in range(n_layers - 1):
            w_r = refs[4 + 3 * l]
            b_r = refs[5 + 3 * l]
            a_r = refs[6 + 3 * l]
            src, dst = srcs[l], acts[l]
            width, cin = widths[l], cins[l]
            sts = (st_big0, st_big1) if l == 0 else (st_map0, st_map1)
            for qi, q0 in enumerate(range(0, width, ck)):
                c = min(ck, width - q0)
                if len(taps[l]) == 1:
                    acc = jnp.dot(w_r[...], src[:, q0:q0 + c],
                                  preferred_element_type=jnp.float32)
                else:
                    st = sts[qi % 2]
                    for t, s in enumerate(taps[l]):
                        st[t * cin:(t + 1) * cin, :c] = src[:, s + q0:s + q0 + c]
                    acc = jnp.dot(w_r[...], st[:, :c],
                                  preferred_element_type=jnp.float32)
                acc = acc + b_r[...]
                acc = jnp.where(acc >= 0.0, acc, acc * a_r[...])
                m = mcat[:, origins[l] + q0:origins[l] + q0 + c]
                acc = jnp.where(m > 0.0, acc, 0.0)
                dst[:, q0:q0 + c] = acc.astype(dst.dtype)

        l = n_layers - 1
        w_r = refs[4 + 3 * l]
        b_r = refs[5 + 3 * l]
        src = srcs[l]
        co = couts[l]
        out[...] = jnp.broadcast_to(b_r[...], (co, tm))
        in_w = tm + max(taps[l])
        for pi, p0 in enumerate(range(0, in_w, ck)):
            c = min(ck, in_w - p0)
            part = (part0, part1)[pi % 2]
            part[:, :c] = jnp.dot(w_r[...], src[:, p0:p0 + c],
                                  preferred_element_type=jnp.float32
                                  ).astype(jnp.bfloat16)
            for t, s in enumerate(taps[l]):
                lo = max(0, s - p0)
                hi = min(c, tm + s - p0)
                if hi > lo:
                    o0 = p0 + lo - s
                    out[:, o0:o0 + hi - lo] = (out[:, o0:o0 + hi - lo]
                                               + part[t * co:(t + 1) * co, lo:hi])
    return body


def kernel(x, w0, b0, a0, w1, b1, a1, w2, b2, a2, w3, b3, a3,
           w4, b4, a4, w5, b5, a5, w6, b6, a6, w7, b7, a7):
    ws = (w0, w1, w2, w3, w4, w5, w6, w7)
    bs = (b0, b1, b2, b3, b4, b5, b6, b7)
    als = (a0, a1, a2, a3, a4, a5, a6, a7)
    N, C, H, W = x.shape
    n_layers = len(ws)

    ksz = [int(round(w.shape[0] ** 0.5)) for w in ws]
    couts = [w.shape[1] for w in ws]
    cin0 = ws[0].shape[2]

    B = max(max(p, k - 1 - p) for p, k in zip(_PADS, ksz))
    Hp, Wp = H + 2 * B, W + 2 * B
    Mp = N * Hp * Wp

    leads = [p * (Wp + 1) for p in _PADS]
    shift = sum(leads)
    tm, ck = _TM, _CK
    assert tm >= 2 * shift + 1044

    widths = [0] * n_layers
    widths[-1] = tm
    for l in range(n_layers - 2, -1, -1):
        widths[l] = _ru(widths[l + 1] + 2 * leads[l + 1], 256)
    assert widths[0] + 2 * leads[0] <= 2 * tm
    origins = [0] * n_layers
    origins[0] = leads[0]
    for l in range(1, n_layers):
        origins[l] = origins[l - 1] + leads[l]

    taps = [tuple(ky * Wp + kx for ky in range(k) for kx in range(k))
            for k in ksz]

    nt = _ru((Mp + tm - 1) // tm, 2)
    lp = (nt + 1) * tm

    xp = jnp.pad(x.astype(jnp.bfloat16),
                 ((0, 0), (0, cin0 - C), (B, B), (B, B)))
    plane = jnp.transpose(xp, (1, 0, 2, 3)).reshape(cin0, Mp)
    x_pad = jnp.pad(plane, ((0, 0), (shift, lp - shift - Mp)))

    idx = jnp.arange(lp, dtype=jnp.int32) - shift
    rem = idx % (Hp * Wp)
    yy, xx = rem // Wp, rem % Wp
    valid = ((idx >= 0) & (idx < Mp) & (yy >= B) & (yy < B + H)
             & (xx >= B) & (xx < B + W))
    mask_p = valid.astype(jnp.float32).reshape(1, lp)

    in_specs = [
        pl.BlockSpec((cin0, tm), lambda j: (0, j)),
        pl.BlockSpec((cin0, tm), lambda j: (0, j + 1)),
        pl.BlockSpec((1, tm), lambda j: (0, j)),
        pl.BlockSpec((1, tm), lambda j: (0, j + 1)),
    ]
    cins = [w.shape[2] for w in ws]
    args = [x_pad, x_pad, mask_p, mask_p]
    for l in range(n_layers):
        if l < n_layers - 1:
            wp = jnp.transpose(ws[l], (1, 0, 2)).reshape(couts[l], -1)
        else:
            wp = ws[l].reshape(-1, cins[l])
        in_specs.append(pl.BlockSpec(wp.shape, lambda j: (0, 0)))
        in_specs.append(pl.BlockSpec(bs[l].shape, lambda j: (0, 0)))
        in_specs.append(pl.BlockSpec(als[l].shape, lambda j: (0, 0)))
        args += [wp, bs[l], als[l]]

    scratch = [pltpu.VMEM((cin0, 2 * tm), jnp.bfloat16),
               pltpu.VMEM((1, 2 * tm), jnp.float32)]
    for l in range(n_layers - 1):
        scratch.append(pltpu.VMEM((couts[l], widths[l]), jnp.bfloat16))
    for _ in range(2):
        scratch.append(pltpu.VMEM((len(taps[0]) * cins[0], ck), jnp.bfloat16))
    for _ in range(2):
        scratch.append(pltpu.VMEM((len(taps[2]) * cins[2], ck), jnp.bfloat16))
    for _ in range(2):
        scratch.append(pltpu.VMEM((len(taps[-1]) * couts[-1], ck), jnp.bfloat16))

    body = _make_body(tm, ck, taps, widths, origins, couts, cins)
    y = pl.pallas_call(
        body,
        out_shape=jax.ShapeDtypeStruct((couts[-1], nt * tm), jnp.float32),
        grid_spec=pltpu.PrefetchScalarGridSpec(
            num_scalar_prefetch=0,
            grid=(nt,),
            in_specs=in_specs,
            out_specs=pl.BlockSpec((couts[-1], tm), lambda j: (0, j)),
            scratch_shapes=scratch,
        ),
        compiler_params=pltpu.CompilerParams(
            dimension_semantics=("arbitrary",),
            vmem_limit_bytes=58 * 1024 * 1024),
    )(*args)

    s, co = 2, 3
    y = y[:s * s * co, :Mp].reshape(s * s * co, N, Hp, Wp)
    y = y[:, :, B:B + H, B:B + W].reshape(s, s, co, N, H, W)
    return jnp.transpose(y, (3, 2, 4, 0, 5, 1)).reshape(N, co, s * H, s * W)


# speedup vs baseline: 1.0477x; 1.0477x over previous
import jax
import jax.numpy as jnp
from jax.experimental import pallas as pl
from jax.experimental.pallas import tpu as pltpu

_TM = 16384
_CK = 4096
_PADS = (2, 0, 1, 1, 1, 1, 0, 2)


def _ru(x, m):
    return (x + m - 1) // m * m


def _make_body(tm, ck, taps, widths, origins, couts, cins):
    n_layers = len(taps)

    def body(*refs):
        xa, xb, ma, mb = refs[0:4]
        out = refs[4 + 3 * n_layers]
        scratch = refs[5 + 3 * n_layers:]
        xcat, mcat = scratch[0], scratch[1]
        acts = scratch[2:2 + n_layers - 1]
        stp = scratch[2 + n_layers - 1:]
        st_of = {0: (stp[0], stp[1]), 2: (stp[2], stp[3]), 3: (stp[4], stp[5]),
                 4: (stp[6], stp[7]), 5: (stp[8], stp[9])}
        parts = (stp[10], stp[11])

        xcat[:, :tm] = xa[...]
        xcat[:, tm:] = xb[...]
        mcat[:, :tm] = ma[...]
        mcat[:, tm:] = mb[...]
        srcs = [xcat] + list(acts)
        ld = n_layers - 1
        co = couts[ld]
        out[...] = jnp.broadcast_to(refs[5 + 3 * ld][...], (co, tm))

        def emit_conv(l, qi, q0):
            w_r, b_r, a_r = refs[4 + 3 * l], refs[5 + 3 * l], refs[6 + 3 * l]
            src, cin = srcs[l], cins[l]
            c = min(ck, widths[l] - q0)
            if len(taps[l]) == 1:
                acc = jnp.dot(w_r[...], src[:, q0:q0 + c],
                              preferred_element_type=jnp.float32)
            else:
                st = st_of[l][qi % 2]
                for t, s in enumerate(taps[l]):
                    st[t * cin:(t + 1) * cin, :c] = src[:, s + q0:s + q0 + c]
                acc = jnp.dot(w_r[...], st[:, :c],
                              preferred_element_type=jnp.float32)
            acc = acc + b_r[...]
            acc = jnp.where(acc >= 0.0, acc, acc * a_r[...])
            m = mcat[:, origins[l] + q0:origins[l] + q0 + c]
            acc = jnp.where(m > 0.0, acc, 0.0)
            acts[l][:, q0:q0 + c] = acc.astype(acts[l].dtype)

        def emit_dec(pi, p0):
            c = min(ck, in_w - p0)
            part = parts[pi % 2]
            part[:, :c] = jnp.dot(refs[4 + 3 * ld][...], srcs[ld][:, p0:p0 + c],
                                  preferred_element_type=jnp.float32
                                  ).astype(jnp.bfloat16)
            for t, s in enumerate(taps[ld]):
                lo = max(0, s - p0)
                hi = min(c, tm + s - p0)
                if hi > lo:
                    o0 = p0 + lo - s
                    out[:, o0:o0 + hi - lo] = (out[:, o0:o0 + hi - lo]
                                               + part[t * co:(t + 1) * co, lo:hi])

        in_w = tm + max(taps[ld])
        chunks = [list(enumerate(range(0, widths[l], ck)))
                  for l in range(n_layers - 1)]
        chunks.append(list(enumerate(range(0, in_w, ck))))
        delay = 2
        n_steps = max(len(chunks[l]) + delay * l for l in range(n_layers))
        for s in range(n_steps):
            for l in range(n_layers):
                q = s - delay * l
                if 0 <= q < len(chunks[l]):
                    qi, q0 = chunks[l][q]
                    if l == ld:
                        emit_dec(qi, q0)
                    else:
                        emit_conv(l, qi, q0)
    return body


def kernel(x, w0, b0, a0, w1, b1, a1, w2, b2, a2, w3, b3, a3,
           w4, b4, a4, w5, b5, a5, w6, b6, a6, w7, b7, a7):
    ws = (w0, w1, w2, w3, w4, w5, w6, w7)
    bs = (b0, b1, b2, b3, b4, b5, b6, b7)
    als = (a0, a1, a2, a3, a4, a5, a6, a7)
    N, C, H, W = x.shape
    n_layers = len(ws)

    ksz = [int(round(w.shape[0] ** 0.5)) for w in ws]
    couts = [w.shape[1] for w in ws]
    cin0 = ws[0].shape[2]

    B = max(max(p, k - 1 - p) for p, k in zip(_PADS, ksz))
    Hp, Wp = H + 2 * B, W + 2 * B
    Mp = N * Hp * Wp

    leads = [p * (Wp + 1) for p in _PADS]
    shift = sum(leads)
    tm, ck = _TM, _CK
    assert tm >= 2 * shift + 1044

    widths = [0] * n_layers
    widths[-1] = tm
    for l in range(n_layers - 2, -1, -1):
        widths[l] = _ru(widths[l + 1] + 2 * leads[l + 1], 256)
    assert widths[0] + 2 * leads[0] <= 2 * tm
    origins = [0] * n_layers
    origins[0] = leads[0]
    for l in range(1, n_layers):
        origins[l] = origins[l - 1] + leads[l]

    taps = [tuple(ky * Wp + kx for ky in range(k) for kx in range(k))
            for k in ksz]

    nt = _ru((Mp + tm - 1) // tm, 2)
    lp = (nt + 1) * tm

    xp = jnp.pad(x.astype(jnp.bfloat16),
                 ((0, 0), (0, cin0 - C), (B, B), (B, B)))
    plane = jnp.transpose(xp, (1, 0, 2, 3)).reshape(cin0, Mp)
    x_pad = jnp.pad(plane, ((0, 0), (shift, lp - shift - Mp)))

    idx = jnp.arange(lp, dtype=jnp.int32) - shift
    rem = idx % (Hp * Wp)
    yy, xx = rem // Wp, rem % Wp
    valid = ((idx >= 0) & (idx < Mp) & (yy >= B) & (yy < B + H)
             & (xx >= B) & (xx < B + W))
    mask_p = valid.astype(jnp.float32).reshape(1, lp)

    in_specs = [
        pl.BlockSpec((cin0, tm), lambda j: (0, j)),
        pl.BlockSpec((cin0, tm), lambda j: (0, j + 1)),
        pl.BlockSpec((1, tm), lambda j: (0, j)),
        pl.BlockSpec((1, tm), lambda j: (0, j + 1)),
    ]
    cins = [w.shape[2] for w in ws]
    args = [x_pad, x_pad, mask_p, mask_p]
    for l in range(n_layers):
        if l < n_layers - 1:
            wp = jnp.transpose(ws[l], (1, 0, 2)).reshape(couts[l], -1)
        else:
            wp = ws[l].reshape(-1, cins[l])
        in_specs.append(pl.BlockSpec(wp.shape, lambda j: (0, 0)))
        in_specs.append(pl.BlockSpec(bs[l].shape, lambda j: (0, 0)))
        in_specs.append(pl.BlockSpec(als[l].shape, lambda j: (0, 0)))
        args += [wp, bs[l], als[l]]

    scratch = [pltpu.VMEM((cin0, 2 * tm), jnp.bfloat16),
               pltpu.VMEM((1, 2 * tm), jnp.float32)]
    for l in range(n_layers - 1):
        scratch.append(pltpu.VMEM((couts[l], widths[l]), jnp.bfloat16))
    for _ in range(2):
        scratch.append(pltpu.VMEM((len(taps[0]) * cins[0], ck), jnp.bfloat16))
    for _ in range(4 * 2):
        scratch.append(pltpu.VMEM((len(taps[2]) * cins[2], ck), jnp.bfloat16))
    for _ in range(2):
        scratch.append(pltpu.VMEM((len(taps[-1]) * couts[-1], ck), jnp.bfloat16))

    body = _make_body(tm, ck, taps, widths, origins, couts, cins)
    y = pl.pallas_call(
        body,
        out_shape=jax.ShapeDtypeStruct((couts[-1], nt * tm), jnp.float32),
        grid_spec=pltpu.PrefetchScalarGridSpec(
            num_scalar_prefetch=0,
            grid=(nt,),
            in_specs=in_specs,
            out_specs=pl.BlockSpec((couts[-1], tm), lambda j: (0, j)),
            scratch_shapes=scratch,
        ),
        compiler_params=pltpu.CompilerParams(
            dimension_semantics=("arbitrary",),
            vmem_limit_bytes=58 * 1024 * 1024),
    )(*args)

    s, co = 2, 3
    y = y[:s * s * co, :Mp].reshape(s * s * co, N, Hp, Wp)
    y = y[:, :, B:B + H, B:B + W].reshape(s, s, co, N, H, W)
    return jnp.transpose(y, (3, 2, 4, 0, 5, 1)).reshape(N, co, s * H, s * W)


# speedup vs baseline: 1.3042x; 1.2448x over previous
import jax
import jax.numpy as jnp
from jax.experimental import pallas as pl
from jax.experimental.pallas import tpu as pltpu

_TM = 16384
_CK = 4096
_PADS = (2, 0, 1, 1, 1, 1, 0, 2)


def _ru(x, m):
    return (x + m - 1) // m * m


def _make_body(tm, ck, taps, widths, origins, couts, cins):
    n_layers = len(taps)

    def body(*refs):
        xa, xb, ma, mb = refs[0:4]
        out = refs[4 + 3 * n_layers]
        scratch = refs[5 + 3 * n_layers:]
        xcat, mcat = scratch[0], scratch[1]
        acts = scratch[2:2 + n_layers - 1]
        stp = scratch[2 + n_layers - 1:]
        st_of = {0: (stp[0], stp[1]), 2: (stp[2], stp[3]), 3: (stp[4], stp[5]),
                 4: (stp[6], stp[7]), 5: (stp[8], stp[9])}
        parts = (stp[10], stp[11])
        oacc = stp[12]

        xcat[:, :tm] = xa[...]
        xcat[:, tm:] = xb[...]
        mcat[:, :tm] = ma[...]
        mcat[:, tm:] = mb[...]
        srcs = [xcat] + list(acts)
        ld = n_layers - 1
        co = couts[ld]
        oacc[...] = jnp.broadcast_to(refs[5 + 3 * ld][...], (co, tm))

        def emit_conv(l, qi, q0):
            w_r, b_r, a_r = refs[4 + 3 * l], refs[5 + 3 * l], refs[6 + 3 * l]
            src, cin = srcs[l], cins[l]
            c = min(ck, widths[l] - q0)
            if len(taps[l]) == 1:
                acc = jnp.dot(w_r[...], src[:, q0:q0 + c],
                              preferred_element_type=jnp.float32)
            else:
                st = st_of[l][qi % 2]
                for t, s in enumerate(taps[l]):
                    st[t * cin:(t + 1) * cin, :c] = src[:, s + q0:s + q0 + c]
                acc = jnp.dot(w_r[...], st[:, :c],
                              preferred_element_type=jnp.float32)
            acc = acc + b_r[...]
            acc = jnp.where(acc >= 0.0, acc, acc * a_r[...])
            m = mcat[:, origins[l] + q0:origins[l] + q0 + c]
            acc = jnp.where(m > 0.0, acc, 0.0)
            acts[l][:, q0:q0 + c] = acc.astype(acts[l].dtype)

        def emit_dec(pi, p0):
            c = min(ck, in_w - p0)
            part = parts[pi % 2]
            part[:, :c] = jnp.dot(refs[4 + 3 * ld][...], srcs[ld][:, p0:p0 + c],
                                  preferred_element_type=jnp.float32
                                  ).astype(jnp.bfloat16)
            for t, s in enumerate(taps[ld]):
                lo = max(0, s - p0)
                hi = min(c, tm + s - p0)
                if hi > lo:
                    o0 = p0 + lo - s
                    oacc[:, o0:o0 + hi - lo] = (oacc[:, o0:o0 + hi - lo]
                                                + part[t * co:(t + 1) * co, lo:hi])

        in_w = tm + max(taps[ld])
        chunks = [list(enumerate(range(0, widths[l], ck)))
                  for l in range(n_layers - 1)]
        chunks.append(list(enumerate(range(0, in_w, ck))))
        delay = 2
        n_steps = max(len(chunks[l]) + delay * l for l in range(n_layers))
        for s in range(n_steps):
            for l in range(n_layers):
                q = s - delay * l
                if 0 <= q < len(chunks[l]):
                    qi, q0 = chunks[l][q]
                    if l == ld:
                        emit_dec(qi, q0)
                    else:
                        emit_conv(l, qi, q0)
        out[...] = oacc[...].astype(out.dtype)
    return body


def kernel(x, w0, b0, a0, w1, b1, a1, w2, b2, a2, w3, b3, a3,
           w4, b4, a4, w5, b5, a5, w6, b6, a6, w7, b7, a7):
    ws = (w0, w1, w2, w3, w4, w5, w6, w7)
    bs = (b0, b1, b2, b3, b4, b5, b6, b7)
    als = (a0, a1, a2, a3, a4, a5, a6, a7)
    N, C, H, W = x.shape
    n_layers = len(ws)

    ksz = [int(round(w.shape[0] ** 0.5)) for w in ws]
    couts = [w.shape[1] for w in ws]
    cin0 = ws[0].shape[2]

    B = max(max(p, k - 1 - p) for p, k in zip(_PADS, ksz))
    Hp, Wp = H + 2 * B, W + 2 * B
    Mp = N * Hp * Wp

    leads = [p * (Wp + 1) for p in _PADS]
    shift = sum(leads)
    tm, ck = _TM, _CK
    assert tm >= 2 * shift + 1044

    widths = [0] * n_layers
    widths[-1] = tm
    for l in range(n_layers - 2, -1, -1):
        widths[l] = _ru(widths[l + 1] + 2 * leads[l + 1], 256)
    assert widths[0] + 2 * leads[0] <= 2 * tm
    origins = [0] * n_layers
    origins[0] = leads[0]
    for l in range(1, n_layers):
        origins[l] = origins[l - 1] + leads[l]

    taps = [tuple(ky * Wp + kx for ky in range(k) for kx in range(k))
            for k in ksz]

    nt = _ru((Mp + tm - 1) // tm, 2)
    lp = (nt + 1) * tm

    xp = jnp.pad(x.astype(jnp.bfloat16),
                 ((0, 0), (0, cin0 - C), (B, B), (B, B)))
    plane = jnp.transpose(xp, (1, 0, 2, 3)).reshape(cin0, Mp)
    x_pad = jnp.pad(plane, ((0, 0), (shift, lp - shift - Mp)))

    idx = jnp.arange(lp, dtype=jnp.int32) - shift
    rem = idx % (Hp * Wp)
    yy, xx = rem // Wp, rem % Wp
    valid = ((idx >= 0) & (idx < Mp) & (yy >= B) & (yy < B + H)
             & (xx >= B) & (xx < B + W))
    mask_p = valid.astype(jnp.bfloat16).reshape(1, lp)

    in_specs = [
        pl.BlockSpec((cin0, tm), lambda j: (0, j)),
        pl.BlockSpec((cin0, tm), lambda j: (0, j + 1)),
        pl.BlockSpec((1, tm), lambda j: (0, j)),
        pl.BlockSpec((1, tm), lambda j: (0, j + 1)),
    ]
    cins = [w.shape[2] for w in ws]
    args = [x_pad, x_pad, mask_p, mask_p]
    for l in range(n_layers):
        if l < n_layers - 1:
            wp = jnp.transpose(ws[l], (1, 0, 2)).reshape(couts[l], -1)
        else:
            wp = ws[l].reshape(-1, cins[l])
        in_specs.append(pl.BlockSpec(wp.shape, lambda j: (0, 0)))
        in_specs.append(pl.BlockSpec(bs[l].shape, lambda j: (0, 0)))
        in_specs.append(pl.BlockSpec(als[l].shape, lambda j: (0, 0)))
        args += [wp, bs[l], als[l]]

    scratch = [pltpu.VMEM((cin0, 2 * tm), jnp.bfloat16),
               pltpu.VMEM((1, 2 * tm), jnp.bfloat16)]
    for l in range(n_layers - 1):
        scratch.append(pltpu.VMEM((couts[l], widths[l]), jnp.bfloat16))
    for _ in range(2):
        scratch.append(pltpu.VMEM((len(taps[0]) * cins[0], ck), jnp.bfloat16))
    for _ in range(4 * 2):
        scratch.append(pltpu.VMEM((len(taps[2]) * cins[2], ck), jnp.bfloat16))
    for _ in range(2):
        scratch.append(pltpu.VMEM((len(taps[-1]) * couts[-1], ck), jnp.bfloat16))
    scratch.append(pltpu.VMEM((couts[-1], tm), jnp.float32))

    body = _make_body(tm, ck, taps, widths, origins, couts, cins)
    y = pl.pallas_call(
        body,
        out_shape=jax.ShapeDtypeStruct((couts[-1], nt * tm), jnp.bfloat16),
        grid_spec=pltpu.PrefetchScalarGridSpec(
            num_scalar_prefetch=0,
            grid=(nt,),
            in_specs=in_specs,
            out_specs=pl.BlockSpec((couts[-1], tm), lambda j: (0, j)),
            scratch_shapes=scratch,
        ),
        compiler_params=pltpu.CompilerParams(
            dimension_semantics=("arbitrary",),
            vmem_limit_bytes=58 * 1024 * 1024),
    )(*args)

    s, co = 2, 3
    y = y[:s * s * co, :Mp].reshape(s * s * co, N, Hp, Wp)
    y = y[:, :, B:B + H, B:B + W].reshape(s, s, co, N, H, W)
    y = jnp.transpose(y, (3, 2, 4, 0, 5, 1)).reshape(N, co, s * H, s * W)
    return y.astype(jnp.float32)
